# Initial kernel scaffold; baseline (speedup 1.0000x reference)
import jax
import jax.numpy as jnp
from jax.experimental import pallas as pl


def kernel(x):
    raise NotImplementedError("write your pallas kernel here")


# speedup vs baseline: 6.4087x; 6.4087x over previous
import jax
import jax.numpy as jnp
from jax.experimental import pallas as pl

_K = 25
_PAD = (_K - 1) // 2


def _decomp_body(x_ref, res_ref, mm_ref):
    x = x_ref[0]
    t = x.shape[0]
    front = jnp.broadcast_to(x[0:1], (_PAD, x.shape[1]))
    back = jnp.broadcast_to(x[t - 1:t], (_PAD, x.shape[1]))
    xp = jnp.concatenate([front, x, back], axis=0)
    a2 = xp[:-1] + xp[1:]
    a4 = a2[:-2] + a2[2:]
    a8 = a4[:-4] + a4[4:]
    a16 = a8[:-8] + a8[8:]
    a24 = a16[0:t] + a8[16:16 + t]
    s25 = a24 + xp[24:24 + t]
    mm = s25 * (1.0 / _K)
    res_ref[0] = x - mm
    mm_ref[0] = mm


def kernel(x):
    b, t, c = x.shape
    out = jax.ShapeDtypeStruct((b, t, c), x.dtype)
    grid = (b,)
    spec = pl.BlockSpec((1, t, c), lambda i: (i, 0, 0))
    res, mm = pl.pallas_call(
        _decomp_body,
        grid=grid,
        in_specs=[spec],
        out_specs=(spec, spec),
        out_shape=(out, out),
    )(x)
    return (res, mm)
